# baseline (speedup 1.0000x reference)
import dataclasses
import functools

import jax
import jax.numpy as jnp
from jax import lax
from jax.experimental import pallas as pl
from jax.experimental.pallas import tpu as pltpu
from jax.experimental.pallas import tpu_sc as plsc

_NL, _NE, _N = 32, 8, 14336
_K = 7168
_ROWS = _NL * _NE
_BR = 32

_R_SC = 32
_R_TC = _ROWS - _R_SC
_NW = 32
_RPW = _R_SC // _NW
_L = 16


def _tc_body(x_ref, o_ref, u_ref):
    x = x_ref[...]
    v = jnp.maximum(x, 0.0)
    u = jax.lax.bitcast_convert_type(v, jnp.int32)
    u_ref[...] = u

    lo = jnp.min(u, axis=1, keepdims=True)
    hi = jnp.max(u, axis=1, keepdims=True)

    def cond(carry):
        lo, hi = carry
        return jnp.any(lo < hi)

    def it(carry):
        lo, hi = carry
        mid = lo + (hi - lo) // 2
        cnt = jnp.sum((u_ref[...] <= mid).astype(jnp.int32), axis=1,
                      keepdims=True)
        pred = cnt >= _K
        return jnp.where(pred, lo, mid + 1), jnp.where(pred, mid, hi)

    lo, hi = jax.lax.while_loop(cond, it, (lo, hi))
    o_ref[...] = jnp.where(u_ref[...] <= lo, 0.0, v)


def _tc_part(flat_tc):
    return pl.pallas_call(
        _tc_body,
        grid=(_R_TC // _BR,),
        in_specs=[pl.BlockSpec((_BR, _N), lambda i: (i, 0))],
        out_specs=pl.BlockSpec((_BR, _N), lambda i: (i, 0)),
        out_shape=jax.ShapeDtypeStruct((_R_TC, _N), jnp.float32),
        scratch_shapes=[pltpu.VMEM((_BR, _N), jnp.int32)],
        compiler_params=pltpu.CompilerParams(
            dimension_semantics=("parallel",),
        ),
    )(flat_tc)


def _sc_row(ubuf):
    @plsc.parallel_loop(0, _N, _L, unroll=8,
                        carry=(jnp.full((_L,), 0x7F800000, jnp.int32),
                               jnp.zeros((_L,), jnp.int32)))
    def seed(i, carry):
        lo_v, hi_v = carry
        c = ubuf[pl.ds(i, _L)]
        return jnp.minimum(lo_v, c), jnp.maximum(hi_v, c)

    lo_v, hi_v = seed
    lo, hi = jnp.min(lo_v), jnp.max(hi_v)

    def cond(carry):
        lo, hi = carry
        return lo < hi

    def it(carry):
        lo, hi = carry
        mid = lo + (hi - lo) // 2

        @plsc.parallel_loop(0, _N, _L, unroll=8,
                            carry=jnp.zeros((_L,), jnp.int32))
        def cnt_vec(i, cnt):
            c = ubuf[pl.ds(i, _L)]
            return cnt + jnp.where(c <= mid, 1, 0)

        cnt = jnp.sum(cnt_vec)
        pred = cnt >= _K
        return (jnp.where(pred, lo, mid + 1), jnp.where(pred, mid, hi))

    lo, hi = lax.while_loop(cond, it, (lo, hi))
    return lo


def _sc_part(flat_sc):
    mesh = plsc.VectorSubcoreMesh(core_axis_name="c", subcore_axis_name="s")
    cp = pltpu.CompilerParams()
    if "needs_layout_passes" in pltpu.CompilerParams.__dataclass_fields__:
        cp = dataclasses.replace(cp, needs_layout_passes=False)

    @functools.partial(
        pl.kernel,
        mesh=mesh,
        compiler_params=cp,
        out_type=jax.ShapeDtypeStruct((_R_SC, _N), jnp.float32),
        scratch_types=[
            pltpu.VMEM((_N,), jnp.float32),
            pltpu.VMEM((_N,), jnp.int32),
            pltpu.SemaphoreType.DMA,
        ],
    )
    def sc_kernel(x_hbm, o_hbm, xbuf, ubuf, sem):
        wid = lax.axis_index("s") * 2 + lax.axis_index("c")
        for r in range(_RPW):
            row = wid * _RPW + r
            pltpu.async_copy(x_hbm.at[row], xbuf, sem).wait()

            @plsc.parallel_loop(0, _N, _L, unroll=8)
            def prep(i):
                c = xbuf[pl.ds(i, _L)]
                v = jnp.maximum(c, 0.0)
                ubuf[pl.ds(i, _L)] = jax.lax.bitcast_convert_type(
                    v, jnp.int32)

            t = _sc_row(ubuf)

            @plsc.parallel_loop(0, _N, _L, unroll=8)
            def mask(i):
                c = ubuf[pl.ds(i, _L)]
                z = jnp.where(c <= t, 0, c)
                xbuf[pl.ds(i, _L)] = jax.lax.bitcast_convert_type(
                    z, jnp.float32)
            pltpu.async_copy(xbuf, o_hbm.at[row], sem).wait()

    return sc_kernel(flat_sc)


def kernel(z_loga_expert):
    flat = z_loga_expert.reshape(_ROWS, _N)
    out_sc = _sc_part(flat[_R_TC:])
    out_tc = _tc_part(flat[:_R_TC])
    return jnp.concatenate([out_tc, out_sc], axis=0).reshape(_NL, _NE, _N)


# speedup vs baseline: 1.8004x; 1.8004x over previous
import jax
import jax.numpy as jnp
from jax.experimental import pallas as pl
from jax.experimental.pallas import tpu as pltpu

_NL, _NE, _N = 32, 8, 14336
_K = 7168
_ROWS = _NL * _NE
_BR = 32
_STATIC_ITERS = 21


def _body(x_ref, o_ref):
    x = x_ref[...]

    mn = jnp.min(x, axis=1, keepdims=True)
    mx = jnp.max(x, axis=1, keepdims=True)
    lo = jax.lax.bitcast_convert_type(jnp.maximum(mn, 0.0), jnp.int32)
    hi = jax.lax.bitcast_convert_type(jnp.maximum(mx, 0.0), jnp.int32)

    def step(carry):
        lo, hi = carry
        mid = lo + (hi - lo) // 2
        mid_f = jax.lax.bitcast_convert_type(mid, jnp.float32)
        cnt = jnp.sum((x_ref[...] <= mid_f).astype(jnp.int32), axis=1,
                      keepdims=True)
        pred = cnt >= _K
        return jnp.where(pred, lo, mid + 1), jnp.where(pred, mid, hi)

    lo, hi = jax.lax.fori_loop(0, _STATIC_ITERS,
                               lambda _, c: step(c), (lo, hi), unroll=7)
    lo, hi = jax.lax.while_loop(
        lambda c: jnp.any(c[0] < c[1]), step, (lo, hi))

    t_f = jax.lax.bitcast_convert_type(lo, jnp.float32)
    o_ref[...] = jnp.where(x_ref[...] <= t_f, 0.0, x)


def kernel(z_loga_expert):
    flat = z_loga_expert.reshape(_ROWS, _N)
    out = pl.pallas_call(
        _body,
        grid=(_ROWS // _BR,),
        in_specs=[pl.BlockSpec((_BR, _N), lambda i: (i, 0))],
        out_specs=pl.BlockSpec((_BR, _N), lambda i: (i, 0)),
        out_shape=jax.ShapeDtypeStruct((_ROWS, _N), jnp.float32),
        compiler_params=pltpu.CompilerParams(
            dimension_semantics=("arbitrary",),
        ),
    )(flat)
    return out.reshape(_NL, _NE, _N)


# speedup vs baseline: 2.1276x; 1.1817x over previous
import jax
import jax.numpy as jnp
from jax.experimental import pallas as pl
from jax.experimental.pallas import tpu as pltpu

_NL, _NE, _N = 32, 8, 14336
_K = 7168
_ROWS = _NL * _NE
_BR = 64
_STATIC_ITERS = 21


def _body(x_ref, o_ref):
    x = x_ref[...]

    mn = jnp.min(x, axis=1, keepdims=True)
    mx = jnp.max(x, axis=1, keepdims=True)
    lo = jax.lax.bitcast_convert_type(jnp.maximum(mn, 0.0), jnp.int32)
    hi = jax.lax.bitcast_convert_type(jnp.maximum(mx, 0.0), jnp.int32)

    def step(carry):
        lo, hi = carry
        mid = lo + (hi - lo) // 2
        mid_f = jax.lax.bitcast_convert_type(mid, jnp.float32)
        cnt = jnp.sum((x_ref[...] <= mid_f).astype(jnp.int32), axis=1,
                      keepdims=True)
        pred = cnt >= _K
        return jnp.where(pred, lo, mid + 1), jnp.where(pred, mid, hi)

    lo, hi = jax.lax.fori_loop(0, _STATIC_ITERS,
                               lambda _, c: step(c), (lo, hi), unroll=7)
    lo, hi = jax.lax.while_loop(
        lambda c: jnp.any(c[0] < c[1]), step, (lo, hi))

    t_f = jax.lax.bitcast_convert_type(lo, jnp.float32)
    o_ref[...] = jnp.where(x_ref[...] <= t_f, 0.0, x)


def kernel(z_loga_expert):
    flat = z_loga_expert.reshape(_ROWS, _N)
    out = pl.pallas_call(
        _body,
        grid=(_ROWS // _BR,),
        in_specs=[pl.BlockSpec((_BR, _N), lambda i: (i, 0))],
        out_specs=pl.BlockSpec((_BR, _N), lambda i: (i, 0)),
        out_shape=jax.ShapeDtypeStruct((_ROWS, _N), jnp.float32),
        compiler_params=pltpu.CompilerParams(
            dimension_semantics=("arbitrary",),
        ),
    )(flat)
    return out.reshape(_NL, _NE, _N)


# speedup vs baseline: 2.2625x; 1.0634x over previous
import jax
import jax.numpy as jnp
from jax.experimental import pallas as pl
from jax.experimental.pallas import tpu as pltpu

_NL, _NE, _N = 32, 8, 14336
_K = 7168
_ROWS = _NL * _NE
_BR = 128
_STATIC_ITERS = 21


def _body(x_ref, o_ref):
    x = x_ref[...]

    mn = jnp.min(x, axis=1, keepdims=True)
    mx = jnp.max(x, axis=1, keepdims=True)
    lo = jax.lax.bitcast_convert_type(jnp.maximum(mn, 0.0), jnp.int32)
    hi = jax.lax.bitcast_convert_type(jnp.maximum(mx, 0.0), jnp.int32)

    def step(carry):
        lo, hi = carry
        mid = lo + (hi - lo) // 2
        mid_f = jax.lax.bitcast_convert_type(mid, jnp.float32)
        cnt = jnp.sum((x_ref[...] <= mid_f).astype(jnp.int32), axis=1,
                      keepdims=True)
        pred = cnt >= _K
        return jnp.where(pred, lo, mid + 1), jnp.where(pred, mid, hi)

    lo, hi = jax.lax.fori_loop(0, _STATIC_ITERS,
                               lambda _, c: step(c), (lo, hi), unroll=7)
    lo, hi = jax.lax.while_loop(
        lambda c: jnp.any(c[0] < c[1]), step, (lo, hi))

    t_f = jax.lax.bitcast_convert_type(lo, jnp.float32)
    o_ref[...] = jnp.where(x_ref[...] <= t_f, 0.0, x)


def kernel(z_loga_expert):
    flat = z_loga_expert.reshape(_ROWS, _N)
    out = pl.pallas_call(
        _body,
        grid=(_ROWS // _BR,),
        in_specs=[pl.BlockSpec((_BR, _N), lambda i: (i, 0))],
        out_specs=pl.BlockSpec((_BR, _N), lambda i: (i, 0)),
        out_shape=jax.ShapeDtypeStruct((_ROWS, _N), jnp.float32),
        compiler_params=pltpu.CompilerParams(
            dimension_semantics=("arbitrary",),
        ),
    )(flat)
    return out.reshape(_NL, _NE, _N)
